# baseline (speedup 1.0000x reference)
import jax
import jax.numpy as jnp
from jax.experimental import pallas as pl
from jax.experimental.pallas import tpu as pltpu

EPS = 1e-8


def _make_body(S, D, PAIRS, shape):
    dn = (((1,), (0,)), ((), ()))
    HI = jax.lax.Precision.HIGHEST

    def body(emb_ref, w_ref, out_ref, wn_ref, gram_ref):
        bi = pl.program_id(1)

        @pl.when(bi == 0)
        def _():
            for s in range(S):
                ws = w_ref[s]
                m = jnp.maximum(
                    jnp.sqrt(jnp.sum(ws * ws, axis=0, keepdims=True)), EPS
                )
                gram_ref[s : s + 1, :] = m
                wn_ref[s] = ws * (1.0 / m)
            for p, (s, t) in enumerate(PAIRS):
                gram_ref[S + p : S + p + 1, :] = jnp.sum(
                    wn_ref[s] * wn_ref[t], axis=0, keepdims=True
                )

        e = emb_ref[...]
        ne = jnp.sqrt(jnp.sum(e * e, axis=1, keepdims=True))
        en = e * (1.0 / jnp.maximum(ne, EPS))

        v1 = jnp.full(shape, -2.0, jnp.float32)
        v2 = jnp.full(shape, -2.0, jnp.float32)
        m1 = jnp.zeros(shape, jnp.float32)
        m2 = jnp.zeros(shape, jnp.float32)
        q1 = jnp.zeros(shape, jnp.int32)
        q2 = jnp.zeros(shape, jnp.int32)
        for s in range(S):
            cos = jax.lax.dot_general(
                en, wn_ref[s], dn, precision=None, preferred_element_type=jnp.float32
            )
            score = cos * 0.5 + (0.5 + EPS)
            mbc = jnp.broadcast_to(gram_ref[s : s + 1, :], shape)
            gt1 = score > v1
            gt2 = score > v2
            v2 = jnp.where(gt1, v1, jnp.where(gt2, score, v2))
            m2 = jnp.where(gt1, m1, jnp.where(gt2, mbc, m2))
            q2 = jnp.where(gt1, q1, jnp.where(gt2, s, q2))
            v1 = jnp.where(gt1, score, v1)
            m1 = jnp.where(gt1, mbc, m1)
            q1 = jnp.where(gt1, s, q1)

        w1 = 1.0 / (1.0 + jnp.exp(v2 - v1))
        w2 = 1.0 - w1
        c1 = 2.0 * (v1 - EPS) - 1.0
        c2 = 2.0 * (v2 - EPS) - 1.0
        lo = jnp.minimum(q1, q2)
        hi = jnp.maximum(q1, q2)
        qq = lo * S + hi
        cross = jnp.zeros(shape, jnp.float32)
        for p, (s, t) in enumerate(PAIRS):
            cross = jnp.where(
                qq == s * S + t,
                jnp.broadcast_to(gram_ref[S + p : S + p + 1, :], shape),
                cross,
            )
        a1 = w1 * m1
        a2 = w2 * m2
        pe = a1 * c1 + a2 * c2
        np2 = a1 * a1 + a2 * a2 + 2.0 * a1 * a2 * cross
        npn = jnp.maximum(jnp.sqrt(jnp.maximum(np2, 0.0)), EPS)
        out_ref[...] = (pe / npn) * 0.5 + (0.5 + EPS)

    return body


def kernel(emb, weight):
    B, D = emb.shape
    C, S, _ = weight.shape
    PAIRS = [(s, t) for s in range(S) for t in range(s + 1, S)]

    cB = 512
    bB = 1024
    CP = ((C + cB - 1) // cB) * cB
    if CP != C:
        weight = jnp.pad(weight, ((0, CP - C), (0, 0), (0, 0)))
    wt = jnp.transpose(weight, (1, 2, 0))

    grid = (CP // cB, B // bB)
    out = pl.pallas_call(
        _make_body(S, D, PAIRS, (bB, cB)),
        grid=grid,
        in_specs=[
            pl.BlockSpec((bB, D), lambda ci, bi: (bi, 0)),
            pl.BlockSpec((S, D, cB), lambda ci, bi: (0, 0, ci)),
        ],
        out_specs=pl.BlockSpec((bB, cB), lambda ci, bi: (bi, ci)),
        out_shape=jax.ShapeDtypeStruct((B, CP), jnp.float32),
        scratch_shapes=[
            pltpu.VMEM((S, D, cB), jnp.float32),
            pltpu.VMEM((S + len(PAIRS), cB), jnp.float32),
        ],
        compiler_params=pltpu.CompilerParams(
            dimension_semantics=("arbitrary", "arbitrary"),
        ),
    )(emb, wt)
    return out[:, :C]


# speedup vs baseline: 1.0668x; 1.0668x over previous
import jax
import jax.numpy as jnp
from jax.experimental import pallas as pl
from jax.experimental.pallas import tpu as pltpu

EPS = 1e-8


def _make_body(S, D, PAIRS, shape):
    dn = (((1,), (0,)), ((), ()))
    HI = jax.lax.Precision.HIGHEST

    def body(emb_ref, w_ref, out_ref, wn_ref, gram_ref):
        bi = pl.program_id(1)

        @pl.when(bi == 0)
        def _():
            for s in range(S):
                ws = w_ref[s]
                m = jnp.maximum(
                    jnp.sqrt(jnp.sum(ws * ws, axis=0, keepdims=True)), EPS
                )
                gram_ref[s : s + 1, :] = m
                wn_ref[s] = ws * (1.0 / m)
            for p, (s, t) in enumerate(PAIRS):
                gram_ref[S + p : S + p + 1, :] = jnp.sum(
                    wn_ref[s] * wn_ref[t], axis=0, keepdims=True
                )

        e = emb_ref[...]
        ne = jnp.sqrt(jnp.sum(e * e, axis=1, keepdims=True))
        en = e * (1.0 / jnp.maximum(ne, EPS))

        v1 = jnp.full(shape, -2.0, jnp.float32)
        v2 = jnp.full(shape, -2.0, jnp.float32)
        m1 = jnp.zeros(shape, jnp.float32)
        m2 = jnp.zeros(shape, jnp.float32)
        q1 = jnp.zeros(shape, jnp.int32)
        q2 = jnp.zeros(shape, jnp.int32)
        for s in range(S):
            cos = jax.lax.dot_general(
                en, wn_ref[s], dn, precision=None, preferred_element_type=jnp.float32
            )
            mbc = jnp.broadcast_to(gram_ref[s : s + 1, :], shape)
            gt1 = cos > v1
            gt2 = cos > v2
            v2 = jnp.where(gt1, v1, jnp.where(gt2, cos, v2))
            m2 = jnp.where(gt1, m1, jnp.where(gt2, mbc, m2))
            q2 = jnp.where(gt1, q1, jnp.where(gt2, s, q2))
            v1 = jnp.where(gt1, cos, v1)
            m1 = jnp.where(gt1, mbc, m1)
            q1 = jnp.where(gt1, s, q1)

        w1 = 1.0 / (1.0 + jnp.exp(0.5 * (v2 - v1)))
        w2 = 1.0 - w1
        c1 = v1
        c2 = v2
        lo = jnp.minimum(q1, q2)
        hi = jnp.maximum(q1, q2)
        qq = lo * S + hi
        cross = jnp.zeros(shape, jnp.float32)
        for p, (s, t) in enumerate(PAIRS):
            cross = jnp.where(
                qq == s * S + t,
                jnp.broadcast_to(gram_ref[S + p : S + p + 1, :], shape),
                cross,
            )
        a1 = w1 * m1
        a2 = w2 * m2
        pe = a1 * c1 + a2 * c2
        np2 = a1 * a1 + a2 * a2 + 2.0 * a1 * a2 * cross
        npn = jnp.maximum(jnp.sqrt(jnp.maximum(np2, 0.0)), EPS)
        out_ref[...] = (pe / npn) * 0.5 + (0.5 + EPS)

    return body


def kernel(emb, weight):
    B, D = emb.shape
    C, S, _ = weight.shape
    PAIRS = [(s, t) for s in range(S) for t in range(s + 1, S)]

    cB = 512
    bB = 512
    CP = ((C + cB - 1) // cB) * cB
    if CP != C:
        weight = jnp.pad(weight, ((0, CP - C), (0, 0), (0, 0)))
    wt = jnp.transpose(weight, (1, 2, 0))

    grid = (CP // cB, B // bB)
    out = pl.pallas_call(
        _make_body(S, D, PAIRS, (bB, cB)),
        grid=grid,
        in_specs=[
            pl.BlockSpec((bB, D), lambda ci, bi: (bi, 0)),
            pl.BlockSpec((S, D, cB), lambda ci, bi: (0, 0, ci)),
        ],
        out_specs=pl.BlockSpec((bB, cB), lambda ci, bi: (bi, ci)),
        out_shape=jax.ShapeDtypeStruct((B, CP), jnp.float32),
        scratch_shapes=[
            pltpu.VMEM((S, D, cB), jnp.float32),
            pltpu.VMEM((S + len(PAIRS), cB), jnp.float32),
        ],
        compiler_params=pltpu.CompilerParams(
            dimension_semantics=("arbitrary", "arbitrary"),
        ),
    )(emb, wt)
    return out[:, :C]
